# baseline (device time: 112656 ns/iter reference)
import jax
import jax.numpy as jnp
from jax import lax
from jax.experimental import pallas as pl
from jax.experimental.pallas import tpu as pltpu

N_DEV = 4
B_SH = 64
D = 2048
H_SH = 4096
B = N_DEV * B_SH
KT = 1024
N_K = H_SH // KT
N_G = 3 * N_K


def kernel(x, Win0, Wout0, Win1, Wout1, Win2, Wout2):
    def body(x_ref, win0, wout0, win1, wout1, win2, wout2, out_ref,
             partial, comm_ag, comm_rs, win_st, wout_st,
             ag_s, ag_r, rs_s, rs_r, win_sems, wout_sems):
        my = lax.axis_index("i")
        left = (my - 1) % N_DEV
        right = (my + 1) % N_DEV
        diag = (my + 2) % N_DEV

        barrier = pltpu.get_barrier_semaphore()
        for nbr in (left, right):
            pl.semaphore_signal(barrier, inc=1, device_id=(nbr,),
                                device_id_type=pl.DeviceIdType.MESH)
        pl.semaphore_wait(barrier, 2)

        wins = [win0, win1, win2]
        wouts = [wout0, wout1, wout2]
        chunk_of_slot = [my, left, right, diag]

        def issue_pair(g):
            l, k = divmod(g, N_K)
            s = g % 2
            cw = pltpu.make_async_copy(
                wins[l].at[:, pl.ds(k * KT, KT)], win_st.at[s], win_sems.at[s])
            cw.start()
            co = pltpu.make_async_copy(
                wouts[l].at[pl.ds(k * KT, KT), :], wout_st.at[s], wout_sems.at[s])
            co.start()
            return (cw, co)

        def wait_pair(pair):
            pair[0].wait()
            pair[1].wait()

        def kg(g, s):
            sl = g % 2
            h = jnp.maximum(
                jnp.dot(comm_ag[s], win_st[sl].astype(jnp.bfloat16),
                        preferred_element_type=jnp.float32),
                0.0).astype(jnp.bfloat16)
            p = jnp.dot(h, wout_st[sl].astype(jnp.bfloat16),
                        preferred_element_type=jnp.float32)
            c = chunk_of_slot[s]
            if g % N_K == 0:
                partial[pl.ds(c * B_SH, B_SH), :] = p
            else:
                partial[pl.ds(c * B_SH, B_SH), :] = (
                    partial[pl.ds(c * B_SH, B_SH), :] + p)

        def pchunk(c):
            return partial[pl.ds(c * B_SH, B_SH), :]

        comm_ag[0] = x_ref[...].astype(jnp.bfloat16)
        pend = {0: issue_pair(0)}

        for l in range(3):
            g0 = l * N_K

            r0R = pltpu.make_async_remote_copy(
                src_ref=comm_ag.at[0], dst_ref=comm_ag.at[1],
                send_sem=ag_s.at[0], recv_sem=ag_r.at[0],
                device_id=(right,), device_id_type=pl.DeviceIdType.MESH)
            r0L = pltpu.make_async_remote_copy(
                src_ref=comm_ag.at[0], dst_ref=comm_ag.at[2],
                send_sem=ag_s.at[1], recv_sem=ag_r.at[1],
                device_id=(left,), device_id_type=pl.DeviceIdType.MESH)
            r0R.start()
            r0L.start()
            wait_pair(pend.pop(g0))
            pend[g0 + 1] = issue_pair(g0 + 1)
            kg(g0, 0)
            r0R.wait()
            r1 = pltpu.make_async_remote_copy(
                src_ref=comm_ag.at[1], dst_ref=comm_ag.at[3],
                send_sem=ag_s.at[2], recv_sem=ag_r.at[2],
                device_id=(right,), device_id_type=pl.DeviceIdType.MESH)
            r1.start()
            kg(g0, 1)
            r0L.wait()
            kg(g0, 2)
            r1.wait()
            kg(g0, 3)

            for k in (1, 2):
                g = g0 + k
                pend[g + 1] = issue_pair(g + 1)
                wait_pair(pend.pop(g))
                for s in range(N_DEV):
                    kg(g, s)

            g = g0 + 3
            if g + 1 < N_G:
                pend[g + 1] = issue_pair(g + 1)
            wait_pair(pend.pop(g))
            kg(g, 3)
            comm_rs[0] = pchunk(diag).astype(jnp.bfloat16)
            r0 = pltpu.make_async_remote_copy(
                src_ref=comm_rs.at[0], dst_ref=comm_rs.at[1],
                send_sem=rs_s.at[0], recv_sem=rs_r.at[0],
                device_id=(right,), device_id_type=pl.DeviceIdType.MESH)
            r0.start()
            kg(g, 2)
            kg(g, 1)
            comm_rs[3] = pchunk(left).astype(jnp.bfloat16)
            r0.wait()
            comm_rs[2] = (comm_rs[1].astype(jnp.float32)
                          + pchunk(right)).astype(jnp.bfloat16)
            r1R = pltpu.make_async_remote_copy(
                src_ref=comm_rs.at[2], dst_ref=comm_rs.at[4],
                send_sem=rs_s.at[1], recv_sem=rs_r.at[1],
                device_id=(right,), device_id_type=pl.DeviceIdType.MESH)
            r1L = pltpu.make_async_remote_copy(
                src_ref=comm_rs.at[3], dst_ref=comm_rs.at[5],
                send_sem=rs_s.at[2], recv_sem=rs_r.at[2],
                device_id=(left,), device_id_type=pl.DeviceIdType.MESH)
            r1R.start()
            r1L.start()
            kg(g, 0)
            r1R.wait()
            r1L.wait()
            result = (pchunk(my) + comm_rs[4].astype(jnp.float32)
                      + comm_rs[5].astype(jnp.float32))

            if l < 2:
                comm_ag[0] = result.astype(jnp.bfloat16)
            else:
                out_ref[...] = result

    return pl.pallas_call(
        body,
        out_shape=jax.ShapeDtypeStruct((B_SH, D), jnp.float32),
        in_specs=[
            pl.BlockSpec(memory_space=pltpu.MemorySpace.VMEM),
            pl.BlockSpec(memory_space=pltpu.MemorySpace.HBM),
            pl.BlockSpec(memory_space=pltpu.MemorySpace.HBM),
            pl.BlockSpec(memory_space=pltpu.MemorySpace.HBM),
            pl.BlockSpec(memory_space=pltpu.MemorySpace.HBM),
            pl.BlockSpec(memory_space=pltpu.MemorySpace.HBM),
            pl.BlockSpec(memory_space=pltpu.MemorySpace.HBM),
        ],
        out_specs=pl.BlockSpec(memory_space=pltpu.MemorySpace.VMEM),
        scratch_shapes=[
            pltpu.VMEM((B, D), jnp.float32),
            pltpu.VMEM((N_DEV, B_SH, D), jnp.bfloat16),
            pltpu.VMEM((6, B_SH, D), jnp.bfloat16),
            pltpu.VMEM((2, D, KT), jnp.float32),
            pltpu.VMEM((2, KT, D), jnp.float32),
            pltpu.SemaphoreType.DMA((3,)),
            pltpu.SemaphoreType.DMA((3,)),
            pltpu.SemaphoreType.DMA((3,)),
            pltpu.SemaphoreType.DMA((3,)),
            pltpu.SemaphoreType.DMA((2,)),
            pltpu.SemaphoreType.DMA((2,)),
        ],
        compiler_params=pltpu.CompilerParams(
            collective_id=0, vmem_limit_bytes=60 * 1024 * 1024),
    )(x, Win0, Wout0, Win1, Wout1, Win2, Wout2)
